# baseline (speedup 1.0000x reference)
import functools

import jax
import jax.numpy as jnp
from jax import lax
from jax.experimental import pallas as pl
from jax.experimental.pallas import tpu as pltpu
from jax.experimental.pallas import tpu_sc as plsc

B = 8
N = 8192
G = 512
K = 32
F = 128
CPAD = 16

NC = 2
NS = 16
NW = NC * NS


def _fps_body(xt_ref, gidx_ref, cx_ref, cy_ref, cz_ref, dist_ref):
    x = xt_ref[:, 0, :]
    y = xt_ref[:, 1, :]
    z = xt_ref[:, 2, :]
    lanes = lax.broadcasted_iota(jnp.int32, (B, N), 1)
    row_off = lax.broadcasted_iota(jnp.int32, (B, 1), 0) * N
    li128 = lax.broadcasted_iota(jnp.int32, (B, 128), 1)
    dist_ref[...] = jnp.full((B, N), 1e10, jnp.float32)

    def outer(j, far_c):
        def step(t, carry):
            far, cx, cy, cz, bidx, bcx, bcy, bcz = carry
            d = (x - cx) ** 2 + (y - cy) ** 2 + (z - cz) ** 2
            dist = jnp.minimum(dist_ref[...], d)
            dist_ref[...] = dist
            sel = li128 == t
            bidx = jnp.where(sel, jnp.broadcast_to(far + row_off, (B, 128)), bidx)
            bcx = jnp.where(sel, jnp.broadcast_to(cx, (B, 128)), bcx)
            bcy = jnp.where(sel, jnp.broadcast_to(cy, (B, 128)), bcy)
            bcz = jnp.where(sel, jnp.broadcast_to(cz, (B, 128)), bcz)
            v, ii, px, py, pz = dist, lanes, x, y, z
            n = N
            while n > 1:
                h = n // 2
                take = v[:, h:] > v[:, :h]
                v = jnp.where(take, v[:, h:], v[:, :h])
                ii = jnp.where(take, ii[:, h:], ii[:, :h])
                px = jnp.where(take, px[:, h:], px[:, :h])
                py = jnp.where(take, py[:, h:], py[:, :h])
                pz = jnp.where(take, pz[:, h:], pz[:, :h])
                n = h
            return ii, px, py, pz, bidx, bcx, bcy, bcz

        far_c = lax.fori_loop(0, 128, step, far_c)
        far, cx, cy, cz, bidx, bcx, bcy, bcz = far_c
        off = pl.multiple_of(j * 128, 128)
        gidx_ref[:, pl.ds(off, 128)] = bidx
        cx_ref[:, pl.ds(off, 128)] = bcx
        cy_ref[:, pl.ds(off, 128)] = bcy
        cz_ref[:, pl.ds(off, 128)] = bcz
        zi = jnp.zeros((B, 128), jnp.int32)
        zf = jnp.zeros((B, 128), jnp.float32)
        return far, cx, cy, cz, zi, zf, zf, zf

    zi = jnp.zeros((B, 128), jnp.int32)
    zf = jnp.zeros((B, 128), jnp.float32)
    init = (jnp.zeros((B, 1), jnp.int32), x[:, 0:1], y[:, 0:1], z[:, 0:1],
            zi, zf, zf, zf)
    lax.fori_loop(0, G // 128, outer, init)


_fps_call = pl.pallas_call(
    _fps_body,
    out_shape=(
        jax.ShapeDtypeStruct((B, G), jnp.int32),
        jax.ShapeDtypeStruct((B, G), jnp.float32),
        jax.ShapeDtypeStruct((B, G), jnp.float32),
        jax.ShapeDtypeStruct((B, G), jnp.float32),
    ),
    scratch_shapes=[pltpu.VMEM((B, N), jnp.float32)],
)


def _knn_body(xt_ref, qx_ref, qy_ref, qz_ref, gidx_ref, s_ref):
    b = pl.program_id(0)
    x = xt_ref[0, 0, :]
    y = xt_ref[0, 1, :]
    z = xt_ref[0, 2, :]
    qx = qx_ref[0, 0, :]
    qy = qy_ref[0, 0, :]
    qz = qz_ref[0, 0, :]

    ci = lax.broadcasted_iota(jnp.int32, (G, 8), 1)
    qxb = jnp.broadcast_to(qx[:, None], (G, 8))
    qyb = jnp.broadcast_to(qy[:, None], (G, 8))
    qzb = jnp.broadcast_to(qz[:, None], (G, 8))
    q = jnp.where(ci == 0, qxb, jnp.where(ci == 1, qyb, jnp.where(ci == 2, qzb, 0.0)))
    ri = lax.broadcasted_iota(jnp.int32, (8, N), 0)
    xb = jnp.broadcast_to(x[None, :], (8, N))
    yb = jnp.broadcast_to(y[None, :], (8, N))
    zb = jnp.broadcast_to(z[None, :], (8, N))
    p = jnp.where(ri == 0, xb, jnp.where(ri == 1, yb, jnp.where(ri == 2, zb, 0.0)))

    mm = lax.dot_general(q, p, (((1,), (0,)), ((), ())),
                         preferred_element_type=jnp.float32)
    qn = (qx * qx + qy * qy) + (qz * qz)
    pn = (x * x + y * y) + (z * z)
    s_ref[...] = (-2.0 * mm + qn[:, None]) + pn[None, :]

    lanes = lax.broadcasted_iota(jnp.int32, (G, N), 1)
    kiota = lax.broadcasted_iota(jnp.int32, (G, K), 1)
    base = b * N

    def rnd(r, ibuf):
        s = s_ref[...]
        v, ii = s, lanes
        n = N
        while n > 128:
            h = n // 2
            take = v[:, h:] < v[:, :h]
            v = jnp.where(take, v[:, h:], v[:, :h])
            ii = jnp.where(take, ii[:, h:], ii[:, :h])
            n = h
        m = jnp.min(v, axis=1, keepdims=True)
        a = jnp.min(jnp.where(v == m, ii, N), axis=1, keepdims=True)
        ibuf = jnp.where(kiota == r, jnp.broadcast_to(a + base, (G, K)), ibuf)
        s_ref[...] = jnp.where(lanes == a, jnp.inf, s)
        return ibuf

    ibuf = lax.fori_loop(0, K, rnd, jnp.zeros((G, K), jnp.int32))
    gidx_ref[0, :, :] = ibuf


_knn_call = pl.pallas_call(
    _knn_body,
    grid=(B,),
    in_specs=[
        pl.BlockSpec((1, 3, N), lambda b: (b, 0, 0)),
        pl.BlockSpec((1, 1, G), lambda b: (b, 0, 0)),
        pl.BlockSpec((1, 1, G), lambda b: (b, 0, 0)),
        pl.BlockSpec((1, 1, G), lambda b: (b, 0, 0)),
    ],
    out_specs=pl.BlockSpec((1, G, K), lambda b: (b, 0, 0)),
    out_shape=jax.ShapeDtypeStruct((B, G, K), jnp.int32),
    scratch_shapes=[pltpu.VMEM((G, N), jnp.float32)],
)


KNN_ROWS = B * G * K
FPS_ROWS = B * G
CHUNK = 128
KNN_CHUNKS_PER_W = KNN_ROWS // (NW * CHUNK)
FPS_CHUNKS_PER_W = FPS_ROWS // (NW * CHUNK)

@functools.lru_cache(maxsize=1)
def _build_gather_call():
    mesh = plsc.VectorSubcoreMesh(core_axis_name="c", subcore_axis_name="s")

    @functools.partial(
        pl.kernel,
        mesh=mesh,
        compiler_params=pltpu.CompilerParams(needs_layout_passes=False),
        out_type=[
            jax.ShapeDtypeStruct((KNN_ROWS, F), jnp.float32),
            jax.ShapeDtypeStruct((KNN_ROWS,), jnp.float32),
            jax.ShapeDtypeStruct((KNN_ROWS,), jnp.float32),
            jax.ShapeDtypeStruct((KNN_ROWS,), jnp.float32),
            jax.ShapeDtypeStruct((FPS_ROWS, F), jnp.float32),
        ],
        scratch_types=[
            pltpu.VMEM((CHUNK,), jnp.int32),
            pltpu.VMEM((CHUNK, F), jnp.float32),
            pltpu.VMEM((N,), jnp.float32),
            pltpu.VMEM((N,), jnp.float32),
            pltpu.VMEM((N,), jnp.float32),
            pltpu.VMEM((CHUNK,), jnp.float32),
            pltpu.VMEM((CHUNK,), jnp.float32),
            pltpu.VMEM((CHUNK,), jnp.float32),
            pltpu.SemaphoreType.DMA,
        ],
    )
    def gather_call(knn_idx_hbm, fps_idx_hbm, feat_hbm, x_hbm, y_hbm, z_hbm,
                    knn_feat_hbm, ox_hbm, oy_hbm, oz_hbm, loc_feat_hbm,
                    idx_v, rows_v, xt_v, yt_v, zt_v, ox_v, oy_v, oz_v, sem):
        w = lax.axis_index("s") * NC + lax.axis_index("c")
        batch = w // (NW // B)
        base = batch * N
        pltpu.sync_copy(x_hbm.at[batch], xt_v)
        pltpu.sync_copy(y_hbm.at[batch], yt_v)
        pltpu.sync_copy(z_hbm.at[batch], zt_v)

        def knn_chunk(j, _):
            chunk = w * KNN_CHUNKS_PER_W + j
            pltpu.sync_copy(knn_idx_hbm.at[chunk], idx_v)
            cp = pltpu.async_copy(feat_hbm.at[idx_v], rows_v, sem)
            for t in range(CHUNK // 16):
                sl = pl.ds(t * 16, 16)
                iv = idx_v[sl] - base
                ox_v[sl] = plsc.load_gather(xt_v, [iv])
                oy_v[sl] = plsc.load_gather(yt_v, [iv])
                oz_v[sl] = plsc.load_gather(zt_v, [iv])
            cp.wait()
            out_sl = pl.ds(chunk * CHUNK, CHUNK)
            pltpu.sync_copy(rows_v, knn_feat_hbm.at[out_sl])
            pltpu.sync_copy(ox_v, ox_hbm.at[out_sl])
            pltpu.sync_copy(oy_v, oy_hbm.at[out_sl])
            pltpu.sync_copy(oz_v, oz_hbm.at[out_sl])
            return 0

        lax.fori_loop(0, KNN_CHUNKS_PER_W, knn_chunk, 0)

        pltpu.sync_copy(fps_idx_hbm.at[w], idx_v)
        pltpu.async_copy(feat_hbm.at[idx_v], rows_v, sem).wait()
        pltpu.sync_copy(rows_v, loc_feat_hbm.at[pl.ds(w * CHUNK, CHUNK)])

    return gather_call


def kernel(point_coordinates, point_features):
    pc = point_coordinates
    pf = point_features
    xt = jnp.transpose(pc, (0, 2, 1))

    gfps, lcx, lcy, lcz = _fps_call(xt)
    gknn = _knn_call(xt, lcx.reshape(B, 1, G), lcy.reshape(B, 1, G),
                     lcz.reshape(B, 1, G))

    feat2d = pf.reshape(B * N, F)
    knn_idx2d = gknn.reshape(KNN_ROWS // CHUNK, CHUNK)
    fps_idx2d = gfps.reshape(FPS_ROWS // CHUNK, CHUNK)

    knn_feat, ox, oy, oz, loc_feat = _build_gather_call()(
        knn_idx2d, fps_idx2d, feat2d, xt[:, 0, :], xt[:, 1, :], xt[:, 2, :])

    local_coordinates = jnp.stack([lcx, lcy, lcz], axis=-1)
    local_features = loc_feat.reshape(B, G, F)
    knn_coordinates = jnp.stack([ox, oy, oz], axis=-1).reshape(B, G, K, 3)
    knn_features = knn_feat.reshape(B, G, K, F)
    return (local_coordinates, local_features, knn_coordinates, knn_features)


# speedup vs baseline: 1.3167x; 1.3167x over previous
import functools

import jax
import jax.numpy as jnp
from jax import lax
from jax.experimental import pallas as pl
from jax.experimental.pallas import tpu as pltpu
from jax.experimental.pallas import tpu_sc as plsc

B = 8
N = 8192
G = 512
K = 32
F = 128
CPAD = 16

NC = 2
NS = 16
NW = NC * NS


def _fps_body(xt_ref, gidx_ref, cx_ref, cy_ref, cz_ref, dist_ref):
    x = xt_ref[:, 0, :]
    y = xt_ref[:, 1, :]
    z = xt_ref[:, 2, :]
    lanes = lax.broadcasted_iota(jnp.int32, (B, N), 1)
    row_off = lax.broadcasted_iota(jnp.int32, (B, 1), 0) * N
    li128 = lax.broadcasted_iota(jnp.int32, (B, 128), 1)
    dist_ref[...] = jnp.full((B, N), 1e10, jnp.float32)

    def outer(j, far):
        def step(t, carry):
            far, bidx, bcx, bcy, bcz = carry
            oh = lanes == far
            cx = jnp.sum(jnp.where(oh, x, 0.0), axis=1, keepdims=True)
            cy = jnp.sum(jnp.where(oh, y, 0.0), axis=1, keepdims=True)
            cz = jnp.sum(jnp.where(oh, z, 0.0), axis=1, keepdims=True)
            d = (x - cx) ** 2 + (y - cy) ** 2 + (z - cz) ** 2
            dist = jnp.minimum(dist_ref[...], d)
            dist_ref[...] = dist
            m = jnp.max(dist, axis=1, keepdims=True)
            cand = jnp.where(dist == m, lanes, N)
            far_new = jnp.min(cand, axis=1, keepdims=True)
            sel = li128 == t
            bidx = jnp.where(sel, jnp.broadcast_to(far + row_off, (B, 128)), bidx)
            bcx = jnp.where(sel, jnp.broadcast_to(cx, (B, 128)), bcx)
            bcy = jnp.where(sel, jnp.broadcast_to(cy, (B, 128)), bcy)
            bcz = jnp.where(sel, jnp.broadcast_to(cz, (B, 128)), bcz)
            return far_new, bidx, bcx, bcy, bcz

        zi = jnp.zeros((B, 128), jnp.int32)
        zf = jnp.zeros((B, 128), jnp.float32)
        far, bidx, bcx, bcy, bcz = lax.fori_loop(0, 128, step, (far, zi, zf, zf, zf))
        off = pl.multiple_of(j * 128, 128)
        gidx_ref[:, pl.ds(off, 128)] = bidx
        cx_ref[:, pl.ds(off, 128)] = bcx
        cy_ref[:, pl.ds(off, 128)] = bcy
        cz_ref[:, pl.ds(off, 128)] = bcz
        return far

    lax.fori_loop(0, G // 128, outer, jnp.zeros((B, 1), jnp.int32))


_fps_call = pl.pallas_call(
    _fps_body,
    out_shape=(
        jax.ShapeDtypeStruct((B, G), jnp.int32),
        jax.ShapeDtypeStruct((B, G), jnp.float32),
        jax.ShapeDtypeStruct((B, G), jnp.float32),
        jax.ShapeDtypeStruct((B, G), jnp.float32),
    ),
    scratch_shapes=[pltpu.VMEM((B, N), jnp.float32)],
)


def _knn_body(xt_ref, qx_ref, qy_ref, qz_ref, gidx_ref, s_ref):
    b = pl.program_id(0)
    x = xt_ref[0, 0, :]
    y = xt_ref[0, 1, :]
    z = xt_ref[0, 2, :]
    qx = qx_ref[0, 0, :]
    qy = qy_ref[0, 0, :]
    qz = qz_ref[0, 0, :]

    ci = lax.broadcasted_iota(jnp.int32, (G, 8), 1)
    qxb = jnp.broadcast_to(qx[:, None], (G, 8))
    qyb = jnp.broadcast_to(qy[:, None], (G, 8))
    qzb = jnp.broadcast_to(qz[:, None], (G, 8))
    q = jnp.where(ci == 0, qxb, jnp.where(ci == 1, qyb, jnp.where(ci == 2, qzb, 0.0)))
    ri = lax.broadcasted_iota(jnp.int32, (8, N), 0)
    xb = jnp.broadcast_to(x[None, :], (8, N))
    yb = jnp.broadcast_to(y[None, :], (8, N))
    zb = jnp.broadcast_to(z[None, :], (8, N))
    p = jnp.where(ri == 0, xb, jnp.where(ri == 1, yb, jnp.where(ri == 2, zb, 0.0)))

    mm = lax.dot_general(q, p, (((1,), (0,)), ((), ())),
                         preferred_element_type=jnp.float32)
    qn = (qx * qx + qy * qy) + (qz * qz)
    pn = (x * x + y * y) + (z * z)
    s_ref[...] = (-2.0 * mm + qn[:, None]) + pn[None, :]

    lanes = lax.broadcasted_iota(jnp.int32, (G, N), 1)
    kiota = lax.broadcasted_iota(jnp.int32, (G, K), 1)
    base = b * N

    def rnd(r, ibuf):
        s = s_ref[...]
        m = jnp.min(s, axis=1, keepdims=True)
        cand = jnp.where(s == m, lanes, N)
        a = jnp.min(cand, axis=1, keepdims=True)
        ibuf = jnp.where(kiota == r, jnp.broadcast_to(a + base, (G, K)), ibuf)
        s_ref[...] = jnp.where(lanes == a, jnp.inf, s)
        return ibuf

    ibuf = lax.fori_loop(0, K, rnd, jnp.zeros((G, K), jnp.int32))
    gidx_ref[0, :, :] = ibuf


_knn_call = pl.pallas_call(
    _knn_body,
    grid=(B,),
    in_specs=[
        pl.BlockSpec((1, 3, N), lambda b: (b, 0, 0)),
        pl.BlockSpec((1, 1, G), lambda b: (b, 0, 0)),
        pl.BlockSpec((1, 1, G), lambda b: (b, 0, 0)),
        pl.BlockSpec((1, 1, G), lambda b: (b, 0, 0)),
    ],
    out_specs=pl.BlockSpec((1, G, K), lambda b: (b, 0, 0)),
    out_shape=jax.ShapeDtypeStruct((B, G, K), jnp.int32),
    scratch_shapes=[pltpu.VMEM((G, N), jnp.float32)],
)


KNN_ROWS = B * G * K
FPS_ROWS = B * G
CHUNK = 128
KNN_CHUNKS_PER_W = KNN_ROWS // (NW * CHUNK)
FPS_CHUNKS_PER_W = FPS_ROWS // (NW * CHUNK)

@functools.lru_cache(maxsize=1)
def _build_gather_call():
    mesh = plsc.VectorSubcoreMesh(core_axis_name="c", subcore_axis_name="s")

    @functools.partial(
        pl.kernel,
        mesh=mesh,
        compiler_params=pltpu.CompilerParams(needs_layout_passes=False),
        out_type=[
            jax.ShapeDtypeStruct((KNN_ROWS, F), jnp.float32),
            jax.ShapeDtypeStruct((KNN_ROWS,), jnp.float32),
            jax.ShapeDtypeStruct((KNN_ROWS,), jnp.float32),
            jax.ShapeDtypeStruct((KNN_ROWS,), jnp.float32),
            jax.ShapeDtypeStruct((FPS_ROWS, F), jnp.float32),
        ],
        scratch_types=[
            pltpu.VMEM((2, CHUNK), jnp.int32),
            pltpu.VMEM((2, CHUNK, F), jnp.float32),
            pltpu.VMEM((N,), jnp.float32),
            pltpu.VMEM((N,), jnp.float32),
            pltpu.VMEM((N,), jnp.float32),
            pltpu.VMEM((2, CHUNK), jnp.float32),
            pltpu.VMEM((2, CHUNK), jnp.float32),
            pltpu.VMEM((2, CHUNK), jnp.float32),
            pltpu.SemaphoreType.DMA,
            pltpu.SemaphoreType.DMA,
            pltpu.SemaphoreType.DMA,
            pltpu.SemaphoreType.DMA,
            pltpu.SemaphoreType.DMA,
            pltpu.SemaphoreType.DMA,
        ],
    )
    def gather_call(knn_idx_hbm, fps_idx_hbm, feat_hbm, x_hbm, y_hbm, z_hbm,
                    knn_feat_hbm, ox_hbm, oy_hbm, oz_hbm, loc_feat_hbm,
                    idx_v, rows_v, xt_v, yt_v, zt_v, ox_v, oy_v, oz_v,
                    sem_i0, sem_i1, sem_g0, sem_g1, sem_w0, sem_w1):
        w = lax.axis_index("s") * NC + lax.axis_index("c")
        batch = w // (NW // B)
        base = batch * N
        pltpu.sync_copy(x_hbm.at[batch], xt_v)
        pltpu.sync_copy(y_hbm.at[batch], yt_v)
        pltpu.sync_copy(z_hbm.at[batch], zt_v)

        sem_i = (sem_i0, sem_i1)
        sem_g = (sem_g0, sem_g1)
        sem_w = (sem_w0, sem_w1)
        nch = KNN_CHUNKS_PER_W
        idx_h = [None, None]
        wb_h = [None, None]
        idx_h[0] = pltpu.async_copy(knn_idx_hbm.at[w * nch], idx_v.at[0],
                                    sem_i[0])
        for j in range(nch):
            cur = j % 2
            nxt = (j + 1) % 2
            if wb_h[cur] is not None:
                for h in wb_h[cur]:
                    h.wait()
            idx_h[cur].wait()
            gh = pltpu.async_copy(feat_hbm.at[idx_v.at[cur]], rows_v.at[cur],
                                  sem_g[cur])
            if j + 1 < nch:
                idx_h[nxt] = pltpu.async_copy(knn_idx_hbm.at[w * nch + j + 1],
                                              idx_v.at[nxt], sem_i[nxt])
            for t in range(CHUNK // 16):
                sl = pl.ds(t * 16, 16)
                iv = idx_v[cur, sl] - base
                ox_v[cur, sl] = plsc.load_gather(xt_v, [iv])
                oy_v[cur, sl] = plsc.load_gather(yt_v, [iv])
                oz_v[cur, sl] = plsc.load_gather(zt_v, [iv])
            gh.wait()
            out_sl = pl.ds((w * nch + j) * CHUNK, CHUNK)
            wb_h[cur] = [
                pltpu.async_copy(rows_v.at[cur], knn_feat_hbm.at[out_sl], sem_w[cur]),
                pltpu.async_copy(ox_v.at[cur], ox_hbm.at[out_sl], sem_w[cur]),
                pltpu.async_copy(oy_v.at[cur], oy_hbm.at[out_sl], sem_w[cur]),
                pltpu.async_copy(oz_v.at[cur], oz_hbm.at[out_sl], sem_w[cur]),
            ]
        for hs in wb_h:
            if hs is not None:
                for h in hs:
                    h.wait()

        pltpu.sync_copy(fps_idx_hbm.at[w], idx_v.at[0])
        pltpu.async_copy(feat_hbm.at[idx_v.at[0]], rows_v.at[0], sem_g0).wait()
        pltpu.sync_copy(rows_v.at[0], loc_feat_hbm.at[pl.ds(w * CHUNK, CHUNK)])

    return gather_call


def kernel(point_coordinates, point_features):
    pc = point_coordinates
    pf = point_features
    xt = jnp.transpose(pc, (0, 2, 1))

    gfps, lcx, lcy, lcz = _fps_call(xt)
    gknn = _knn_call(xt, lcx.reshape(B, 1, G), lcy.reshape(B, 1, G),
                     lcz.reshape(B, 1, G))

    feat2d = pf.reshape(B * N, F)
    knn_idx2d = gknn.reshape(KNN_ROWS // CHUNK, CHUNK)
    fps_idx2d = gfps.reshape(FPS_ROWS // CHUNK, CHUNK)

    knn_feat, ox, oy, oz, loc_feat = _build_gather_call()(
        knn_idx2d, fps_idx2d, feat2d, xt[:, 0, :], xt[:, 1, :], xt[:, 2, :])

    local_coordinates = jnp.stack([lcx, lcy, lcz], axis=-1)
    local_features = loc_feat.reshape(B, G, F)
    knn_coordinates = jnp.stack([ox, oy, oz], axis=-1).reshape(B, G, K, 3)
    knn_features = knn_feat.reshape(B, G, K, F)
    return (local_coordinates, local_features, knn_coordinates, knn_features)


# speedup vs baseline: 1.4761x; 1.1210x over previous
import functools

import jax
import jax.numpy as jnp
from jax import lax
from jax.experimental import pallas as pl
from jax.experimental.pallas import tpu as pltpu
from jax.experimental.pallas import tpu_sc as plsc

B = 8
N = 8192
G = 512
K = 32
F = 128
CPAD = 16

NC = 2
NS = 16
NW = NC * NS


def _fps_body(xt_ref, gidx_ref, cx_ref, cy_ref, cz_ref, dist_ref):
    x = xt_ref[:, 0, :]
    y = xt_ref[:, 1, :]
    z = xt_ref[:, 2, :]
    lanes = lax.broadcasted_iota(jnp.int32, (B, N), 1).astype(jnp.float32)
    row_off = lax.broadcasted_iota(jnp.int32, (B, 1), 0) * N
    li128 = lax.broadcasted_iota(jnp.int32, (B, 128), 1)
    dist_ref[...] = jnp.full((B, N), 1e10, jnp.float32)

    def outer(j, far):
        def step(t, carry):
            far, bidx, bcx, bcy, bcz = carry
            oh = lanes == far.astype(jnp.float32)
            cx = jnp.sum(jnp.where(oh, x, 0.0), axis=1, keepdims=True)
            cy = jnp.sum(jnp.where(oh, y, 0.0), axis=1, keepdims=True)
            cz = jnp.sum(jnp.where(oh, z, 0.0), axis=1, keepdims=True)
            d = (x - cx) ** 2 + (y - cy) ** 2 + (z - cz) ** 2
            dist = jnp.minimum(dist_ref[...], d)
            dist_ref[...] = dist
            m = jnp.max(dist, axis=1, keepdims=True)
            cand = jnp.where(dist == m, lanes, jnp.float32(N))
            far_new = jnp.min(cand, axis=1, keepdims=True).astype(jnp.int32)
            sel = li128 == t
            bidx = jnp.where(sel, jnp.broadcast_to(far + row_off, (B, 128)), bidx)
            bcx = jnp.where(sel, jnp.broadcast_to(cx, (B, 128)), bcx)
            bcy = jnp.where(sel, jnp.broadcast_to(cy, (B, 128)), bcy)
            bcz = jnp.where(sel, jnp.broadcast_to(cz, (B, 128)), bcz)
            return far_new, bidx, bcx, bcy, bcz

        zi = jnp.zeros((B, 128), jnp.int32)
        zf = jnp.zeros((B, 128), jnp.float32)
        far, bidx, bcx, bcy, bcz = lax.fori_loop(0, 128, step, (far, zi, zf, zf, zf))
        off = pl.multiple_of(j * 128, 128)
        gidx_ref[:, pl.ds(off, 128)] = bidx
        cx_ref[:, pl.ds(off, 128)] = bcx
        cy_ref[:, pl.ds(off, 128)] = bcy
        cz_ref[:, pl.ds(off, 128)] = bcz
        return far

    lax.fori_loop(0, G // 128, outer, jnp.zeros((B, 1), jnp.int32))


_fps_call = pl.pallas_call(
    _fps_body,
    out_shape=(
        jax.ShapeDtypeStruct((B, G), jnp.int32),
        jax.ShapeDtypeStruct((B, G), jnp.float32),
        jax.ShapeDtypeStruct((B, G), jnp.float32),
        jax.ShapeDtypeStruct((B, G), jnp.float32),
    ),
    scratch_shapes=[pltpu.VMEM((B, N), jnp.float32)],
)


def _knn_body(xt_ref, qx_ref, qy_ref, qz_ref, gidx_ref, s_ref):
    b = pl.program_id(0)
    x = xt_ref[0, 0, :]
    y = xt_ref[0, 1, :]
    z = xt_ref[0, 2, :]
    qx = qx_ref[0, 0, :]
    qy = qy_ref[0, 0, :]
    qz = qz_ref[0, 0, :]

    ci = lax.broadcasted_iota(jnp.int32, (G, 8), 1)
    qxb = jnp.broadcast_to(qx[:, None], (G, 8))
    qyb = jnp.broadcast_to(qy[:, None], (G, 8))
    qzb = jnp.broadcast_to(qz[:, None], (G, 8))
    q = jnp.where(ci == 0, qxb, jnp.where(ci == 1, qyb, jnp.where(ci == 2, qzb, 0.0)))
    ri = lax.broadcasted_iota(jnp.int32, (8, N), 0)
    xb = jnp.broadcast_to(x[None, :], (8, N))
    yb = jnp.broadcast_to(y[None, :], (8, N))
    zb = jnp.broadcast_to(z[None, :], (8, N))
    p = jnp.where(ri == 0, xb, jnp.where(ri == 1, yb, jnp.where(ri == 2, zb, 0.0)))

    mm = lax.dot_general(q, p, (((1,), (0,)), ((), ())),
                         preferred_element_type=jnp.float32)
    qn = (qx * qx + qy * qy) + (qz * qz)
    pn = (x * x + y * y) + (z * z)
    s_ref[...] = (-2.0 * mm + qn[:, None]) + pn[None, :]

    lanes = lax.broadcasted_iota(jnp.int32, (G, N), 1).astype(jnp.float32)
    kiota = lax.broadcasted_iota(jnp.int32, (G, K), 1)
    base = b * N

    def rnd(r, ibuf):
        s = s_ref[...]
        m = jnp.min(s, axis=1, keepdims=True)
        cand = jnp.where(s == m, lanes, jnp.float32(N))
        a = jnp.min(cand, axis=1, keepdims=True)
        ai = a.astype(jnp.int32)
        ibuf = jnp.where(kiota == r, jnp.broadcast_to(ai + base, (G, K)), ibuf)
        s_ref[...] = jnp.where(lanes == a, jnp.inf, s)
        return ibuf

    ibuf = lax.fori_loop(0, K, rnd, jnp.zeros((G, K), jnp.int32))
    gidx_ref[0, :, :] = ibuf


_knn_call = pl.pallas_call(
    _knn_body,
    grid=(B,),
    in_specs=[
        pl.BlockSpec((1, 3, N), lambda b: (b, 0, 0)),
        pl.BlockSpec((1, 1, G), lambda b: (b, 0, 0)),
        pl.BlockSpec((1, 1, G), lambda b: (b, 0, 0)),
        pl.BlockSpec((1, 1, G), lambda b: (b, 0, 0)),
    ],
    out_specs=pl.BlockSpec((1, G, K), lambda b: (b, 0, 0)),
    out_shape=jax.ShapeDtypeStruct((B, G, K), jnp.int32),
    scratch_shapes=[pltpu.VMEM((G, N), jnp.float32)],
)


KNN_ROWS = B * G * K
FPS_ROWS = B * G
CHUNK = 128
KNN_CHUNKS_PER_W = KNN_ROWS // (NW * CHUNK)
FPS_CHUNKS_PER_W = FPS_ROWS // (NW * CHUNK)

@functools.lru_cache(maxsize=1)
def _build_gather_call():
    mesh = plsc.VectorSubcoreMesh(core_axis_name="c", subcore_axis_name="s")

    @functools.partial(
        pl.kernel,
        mesh=mesh,
        compiler_params=pltpu.CompilerParams(needs_layout_passes=False),
        out_type=[
            jax.ShapeDtypeStruct((KNN_ROWS, F), jnp.float32),
            jax.ShapeDtypeStruct((KNN_ROWS,), jnp.float32),
            jax.ShapeDtypeStruct((KNN_ROWS,), jnp.float32),
            jax.ShapeDtypeStruct((KNN_ROWS,), jnp.float32),
            jax.ShapeDtypeStruct((FPS_ROWS, F), jnp.float32),
        ],
        scratch_types=[
            pltpu.VMEM((2, CHUNK), jnp.int32),
            pltpu.VMEM((2, CHUNK, F), jnp.float32),
            pltpu.VMEM((N,), jnp.float32),
            pltpu.VMEM((N,), jnp.float32),
            pltpu.VMEM((N,), jnp.float32),
            pltpu.VMEM((2, CHUNK), jnp.float32),
            pltpu.VMEM((2, CHUNK), jnp.float32),
            pltpu.VMEM((2, CHUNK), jnp.float32),
            pltpu.SemaphoreType.DMA,
            pltpu.SemaphoreType.DMA,
            pltpu.SemaphoreType.DMA,
            pltpu.SemaphoreType.DMA,
            pltpu.SemaphoreType.DMA,
            pltpu.SemaphoreType.DMA,
        ],
    )
    def gather_call(knn_idx_hbm, fps_idx_hbm, feat_hbm, x_hbm, y_hbm, z_hbm,
                    knn_feat_hbm, ox_hbm, oy_hbm, oz_hbm, loc_feat_hbm,
                    idx_v, rows_v, xt_v, yt_v, zt_v, ox_v, oy_v, oz_v,
                    sem_i0, sem_i1, sem_g0, sem_g1, sem_w0, sem_w1):
        w = lax.axis_index("s") * NC + lax.axis_index("c")
        batch = w // (NW // B)
        base = batch * N
        pltpu.sync_copy(x_hbm.at[batch], xt_v)
        pltpu.sync_copy(y_hbm.at[batch], yt_v)
        pltpu.sync_copy(z_hbm.at[batch], zt_v)

        sem_i = (sem_i0, sem_i1)
        sem_g = (sem_g0, sem_g1)
        sem_w = (sem_w0, sem_w1)
        nch = KNN_CHUNKS_PER_W
        idx_h = [None, None]
        wb_h = [None, None]
        idx_h[0] = pltpu.async_copy(knn_idx_hbm.at[w * nch], idx_v.at[0],
                                    sem_i[0])
        for j in range(nch):
            cur = j % 2
            nxt = (j + 1) % 2
            if wb_h[cur] is not None:
                for h in wb_h[cur]:
                    h.wait()
            idx_h[cur].wait()
            gh = pltpu.async_copy(feat_hbm.at[idx_v.at[cur]], rows_v.at[cur],
                                  sem_g[cur])
            if j + 1 < nch:
                idx_h[nxt] = pltpu.async_copy(knn_idx_hbm.at[w * nch + j + 1],
                                              idx_v.at[nxt], sem_i[nxt])
            for t in range(CHUNK // 16):
                sl = pl.ds(t * 16, 16)
                iv = idx_v[cur, sl] - base
                ox_v[cur, sl] = plsc.load_gather(xt_v, [iv])
                oy_v[cur, sl] = plsc.load_gather(yt_v, [iv])
                oz_v[cur, sl] = plsc.load_gather(zt_v, [iv])
            gh.wait()
            out_sl = pl.ds((w * nch + j) * CHUNK, CHUNK)
            wb_h[cur] = [
                pltpu.async_copy(rows_v.at[cur], knn_feat_hbm.at[out_sl], sem_w[cur]),
                pltpu.async_copy(ox_v.at[cur], ox_hbm.at[out_sl], sem_w[cur]),
                pltpu.async_copy(oy_v.at[cur], oy_hbm.at[out_sl], sem_w[cur]),
                pltpu.async_copy(oz_v.at[cur], oz_hbm.at[out_sl], sem_w[cur]),
            ]
        for hs in wb_h:
            if hs is not None:
                for h in hs:
                    h.wait()

        pltpu.sync_copy(fps_idx_hbm.at[w], idx_v.at[0])
        pltpu.async_copy(feat_hbm.at[idx_v.at[0]], rows_v.at[0], sem_g0).wait()
        pltpu.sync_copy(rows_v.at[0], loc_feat_hbm.at[pl.ds(w * CHUNK, CHUNK)])

    return gather_call


def kernel(point_coordinates, point_features):
    pc = point_coordinates
    pf = point_features
    xt = jnp.transpose(pc, (0, 2, 1))

    gfps, lcx, lcy, lcz = _fps_call(xt)
    gknn = _knn_call(xt, lcx.reshape(B, 1, G), lcy.reshape(B, 1, G),
                     lcz.reshape(B, 1, G))

    feat2d = pf.reshape(B * N, F)
    knn_idx2d = gknn.reshape(KNN_ROWS // CHUNK, CHUNK)
    fps_idx2d = gfps.reshape(FPS_ROWS // CHUNK, CHUNK)

    knn_feat, ox, oy, oz, loc_feat = _build_gather_call()(
        knn_idx2d, fps_idx2d, feat2d, xt[:, 0, :], xt[:, 1, :], xt[:, 2, :])

    local_coordinates = jnp.stack([lcx, lcy, lcz], axis=-1)
    local_features = loc_feat.reshape(B, G, F)
    knn_coordinates = jnp.stack([ox, oy, oz], axis=-1).reshape(B, G, K, 3)
    knn_features = knn_feat.reshape(B, G, K, F)
    return (local_coordinates, local_features, knn_coordinates, knn_features)


# speedup vs baseline: 1.5836x; 1.0729x over previous
import functools

import jax
import jax.numpy as jnp
from jax import lax
from jax.experimental import pallas as pl
from jax.experimental.pallas import tpu as pltpu
from jax.experimental.pallas import tpu_sc as plsc

B = 8
N = 8192
G = 512
K = 32
F = 128
CPAD = 16

NC = 2
NS = 16
NW = NC * NS


def _tc_body(xt_ref, gidx_ref, cx_ref, cy_ref, cz_ref, gknn_ref, dist_ref, s_ref):
    x = xt_ref[:, 0, :]
    y = xt_ref[:, 1, :]
    z = xt_ref[:, 2, :]
    lanes = lax.broadcasted_iota(jnp.int32, (B, N), 1).astype(jnp.float32)
    row_off = lax.broadcasted_iota(jnp.int32, (B, 1), 0) * N
    li128 = lax.broadcasted_iota(jnp.int32, (B, 128), 1)
    dist_ref[...] = jnp.full((B, N), 1e10, jnp.float32)

    def outer(j, far):
        def step(t, carry):
            far, bidx, bcx, bcy, bcz = carry
            oh = lanes == far.astype(jnp.float32)
            cx = jnp.sum(jnp.where(oh, x, 0.0), axis=1, keepdims=True)
            cy = jnp.sum(jnp.where(oh, y, 0.0), axis=1, keepdims=True)
            cz = jnp.sum(jnp.where(oh, z, 0.0), axis=1, keepdims=True)
            d = (x - cx) ** 2 + (y - cy) ** 2 + (z - cz) ** 2
            dist = jnp.minimum(dist_ref[...], d)
            dist_ref[...] = dist
            m = jnp.max(dist, axis=1, keepdims=True)
            cand = jnp.where(dist == m, lanes, jnp.float32(N))
            far_new = jnp.min(cand, axis=1, keepdims=True).astype(jnp.int32)
            sel = li128 == t
            bidx = jnp.where(sel, jnp.broadcast_to(far + row_off, (B, 128)), bidx)
            bcx = jnp.where(sel, jnp.broadcast_to(cx, (B, 128)), bcx)
            bcy = jnp.where(sel, jnp.broadcast_to(cy, (B, 128)), bcy)
            bcz = jnp.where(sel, jnp.broadcast_to(cz, (B, 128)), bcz)
            return far_new, bidx, bcx, bcy, bcz

        zi = jnp.zeros((B, 128), jnp.int32)
        zf = jnp.zeros((B, 128), jnp.float32)
        far, bidx, bcx, bcy, bcz = lax.fori_loop(0, 128, step, (far, zi, zf, zf, zf))
        off = pl.multiple_of(j * 128, 128)
        gidx_ref[:, pl.ds(off, 128)] = bidx
        cx_ref[:, pl.ds(off, 128)] = bcx
        cy_ref[:, pl.ds(off, 128)] = bcy
        cz_ref[:, pl.ds(off, 128)] = bcz
        return far

    lax.fori_loop(0, G // 128, outer, jnp.zeros((B, 1), jnp.int32))

    lanesq = lax.broadcasted_iota(jnp.int32, (G, N), 1).astype(jnp.float32)
    kiota = lax.broadcasted_iota(jnp.int32, (G, K), 1)
    ci = lax.broadcasted_iota(jnp.int32, (G, 8), 1)
    ri = lax.broadcasted_iota(jnp.int32, (8, N), 0)

    for b in range(B):
        xx = xt_ref[b, 0, :]
        yy = xt_ref[b, 1, :]
        zz = xt_ref[b, 2, :]
        qx = cx_ref[b, :]
        qy = cy_ref[b, :]
        qz = cz_ref[b, :]

        qxb = jnp.broadcast_to(qx[:, None], (G, 8))
        qyb = jnp.broadcast_to(qy[:, None], (G, 8))
        qzb = jnp.broadcast_to(qz[:, None], (G, 8))
        q = jnp.where(ci == 0, qxb, jnp.where(ci == 1, qyb, jnp.where(ci == 2, qzb, 0.0)))
        xb = jnp.broadcast_to(xx[None, :], (8, N))
        yb = jnp.broadcast_to(yy[None, :], (8, N))
        zb = jnp.broadcast_to(zz[None, :], (8, N))
        p = jnp.where(ri == 0, xb, jnp.where(ri == 1, yb, jnp.where(ri == 2, zb, 0.0)))

        mm = lax.dot_general(q, p, (((1,), (0,)), ((), ())),
                             preferred_element_type=jnp.float32)
        qn = (qx * qx + qy * qy) + (qz * qz)
        pn = (xx * xx + yy * yy) + (zz * zz)
        s_ref[...] = (-2.0 * mm + qn[:, None]) + pn[None, :]

        base = b * N

        def rnd(r, ibuf):
            s = s_ref[...]
            m = jnp.min(s, axis=1, keepdims=True)
            cand = jnp.where(s == m, lanesq, jnp.float32(N))
            a = jnp.min(cand, axis=1, keepdims=True)
            ai = a.astype(jnp.int32)
            ibuf = jnp.where(kiota == r, jnp.broadcast_to(ai + base, (G, K)), ibuf)
            s_ref[...] = jnp.where(lanesq == a, jnp.inf, s)
            return ibuf

        ibuf = lax.fori_loop(0, K, rnd, jnp.zeros((G, K), jnp.int32))
        gknn_ref[b, :, :] = ibuf


_tc_call = pl.pallas_call(
    _tc_body,
    out_shape=(
        jax.ShapeDtypeStruct((B, G), jnp.int32),
        jax.ShapeDtypeStruct((B, G), jnp.float32),
        jax.ShapeDtypeStruct((B, G), jnp.float32),
        jax.ShapeDtypeStruct((B, G), jnp.float32),
        jax.ShapeDtypeStruct((B, G, K), jnp.int32),
    ),
    scratch_shapes=[pltpu.VMEM((B, N), jnp.float32),
                    pltpu.VMEM((G, N), jnp.float32)],
)


KNN_ROWS = B * G * K
FPS_ROWS = B * G
CHUNK = 128
KNN_CHUNKS_PER_W = KNN_ROWS // (NW * CHUNK)
FPS_CHUNKS_PER_W = FPS_ROWS // (NW * CHUNK)

@functools.lru_cache(maxsize=1)
def _build_gather_call():
    mesh = plsc.VectorSubcoreMesh(core_axis_name="c", subcore_axis_name="s")

    @functools.partial(
        pl.kernel,
        mesh=mesh,
        compiler_params=pltpu.CompilerParams(needs_layout_passes=False),
        out_type=[
            jax.ShapeDtypeStruct((KNN_ROWS, F), jnp.float32),
            jax.ShapeDtypeStruct((KNN_ROWS,), jnp.float32),
            jax.ShapeDtypeStruct((KNN_ROWS,), jnp.float32),
            jax.ShapeDtypeStruct((KNN_ROWS,), jnp.float32),
            jax.ShapeDtypeStruct((FPS_ROWS, F), jnp.float32),
        ],
        scratch_types=[
            pltpu.VMEM((2, CHUNK), jnp.int32),
            pltpu.VMEM((2, CHUNK, F), jnp.float32),
            pltpu.VMEM((N,), jnp.float32),
            pltpu.VMEM((N,), jnp.float32),
            pltpu.VMEM((N,), jnp.float32),
            pltpu.VMEM((2, CHUNK), jnp.float32),
            pltpu.VMEM((2, CHUNK), jnp.float32),
            pltpu.VMEM((2, CHUNK), jnp.float32),
            pltpu.SemaphoreType.DMA,
            pltpu.SemaphoreType.DMA,
            pltpu.SemaphoreType.DMA,
            pltpu.SemaphoreType.DMA,
            pltpu.SemaphoreType.DMA,
            pltpu.SemaphoreType.DMA,
        ],
    )
    def gather_call(knn_idx_hbm, fps_idx_hbm, feat_hbm, x_hbm, y_hbm, z_hbm,
                    knn_feat_hbm, ox_hbm, oy_hbm, oz_hbm, loc_feat_hbm,
                    idx_v, rows_v, xt_v, yt_v, zt_v, ox_v, oy_v, oz_v,
                    sem_i0, sem_i1, sem_g0, sem_g1, sem_w0, sem_w1):
        w = lax.axis_index("s") * NC + lax.axis_index("c")
        batch = w // (NW // B)
        base = batch * N
        pltpu.sync_copy(x_hbm.at[batch], xt_v)
        pltpu.sync_copy(y_hbm.at[batch], yt_v)
        pltpu.sync_copy(z_hbm.at[batch], zt_v)

        sem_i = (sem_i0, sem_i1)
        sem_g = (sem_g0, sem_g1)
        sem_w = (sem_w0, sem_w1)
        nch = KNN_CHUNKS_PER_W
        idx_h = [None, None]
        wb_h = [None, None]
        idx_h[0] = pltpu.async_copy(knn_idx_hbm.at[w * nch], idx_v.at[0],
                                    sem_i[0])
        for j in range(nch):
            cur = j % 2
            nxt = (j + 1) % 2
            if wb_h[cur] is not None:
                for h in wb_h[cur]:
                    h.wait()
            idx_h[cur].wait()
            gh = pltpu.async_copy(feat_hbm.at[idx_v.at[cur]], rows_v.at[cur],
                                  sem_g[cur])
            if j + 1 < nch:
                idx_h[nxt] = pltpu.async_copy(knn_idx_hbm.at[w * nch + j + 1],
                                              idx_v.at[nxt], sem_i[nxt])
            for t in range(CHUNK // 16):
                sl = pl.ds(t * 16, 16)
                iv = idx_v[cur, sl] - base
                ox_v[cur, sl] = plsc.load_gather(xt_v, [iv])
                oy_v[cur, sl] = plsc.load_gather(yt_v, [iv])
                oz_v[cur, sl] = plsc.load_gather(zt_v, [iv])
            gh.wait()
            out_sl = pl.ds((w * nch + j) * CHUNK, CHUNK)
            wb_h[cur] = [
                pltpu.async_copy(rows_v.at[cur], knn_feat_hbm.at[out_sl], sem_w[cur]),
                pltpu.async_copy(ox_v.at[cur], ox_hbm.at[out_sl], sem_w[cur]),
                pltpu.async_copy(oy_v.at[cur], oy_hbm.at[out_sl], sem_w[cur]),
                pltpu.async_copy(oz_v.at[cur], oz_hbm.at[out_sl], sem_w[cur]),
            ]
        for hs in wb_h:
            if hs is not None:
                for h in hs:
                    h.wait()

        pltpu.sync_copy(fps_idx_hbm.at[w], idx_v.at[0])
        pltpu.async_copy(feat_hbm.at[idx_v.at[0]], rows_v.at[0], sem_g0).wait()
        pltpu.sync_copy(rows_v.at[0], loc_feat_hbm.at[pl.ds(w * CHUNK, CHUNK)])

    return gather_call


def kernel(point_coordinates, point_features):
    pc = point_coordinates
    pf = point_features
    xt = jnp.transpose(pc, (0, 2, 1))

    gfps, lcx, lcy, lcz, gknn = _tc_call(xt)

    feat2d = pf.reshape(B * N, F)
    knn_idx2d = gknn.reshape(KNN_ROWS // CHUNK, CHUNK)
    fps_idx2d = gfps.reshape(FPS_ROWS // CHUNK, CHUNK)

    knn_feat, ox, oy, oz, loc_feat = _build_gather_call()(
        knn_idx2d, fps_idx2d, feat2d, xt[:, 0, :], xt[:, 1, :], xt[:, 2, :])

    local_coordinates = jnp.stack([lcx, lcy, lcz], axis=-1)
    local_features = loc_feat.reshape(B, G, F)
    knn_coordinates = jnp.stack([ox, oy, oz], axis=-1).reshape(B, G, K, 3)
    knn_features = knn_feat.reshape(B, G, K, F)
    return (local_coordinates, local_features, knn_coordinates, knn_features)
